# initial kernel scaffold (unmeasured)
import jax
import jax.numpy as jnp
from jax import lax
from jax.experimental import pallas as pl
from jax.experimental.pallas import tpu as pltpu

N_DEV = 16
HQ_PER = 8
DH = 128
SQ = 256
SKV = 4096
DM = 1024
SCALE = 0.08838834764831843
BF = jnp.bfloat16


def _body(x_ref, wq_ref, k_ref, v_ref, wo_ref, out_ref,
          q_ref, bias_ref, ctx_ref, kbuf, vbuf, comm,
          ksem, vsem, send_sems, recv_sems, credit_sem):
    my = lax.axis_index("i")
    left = lax.rem(my + N_DEV - 1, N_DEV)
    right = lax.rem(my + 1, N_DEV)

    q_ref[...] = jnp.dot(x_ref[...].astype(BF), wq_ref[...].astype(BF),
                         preferred_element_type=jnp.float32)

    qb = lax.broadcasted_iota(jnp.int32, (SQ, SKV), 0) // 64
    kb = lax.broadcasted_iota(jnp.int32, (SQ, SKV), 1) // 64
    mask = (qb == kb) | (kb == 0) | (lax.rem(qb + kb, 3) == 0)
    bias_ref[...] = jnp.where(mask, 0.0, -1e9)

    def start_kv(h, slot):
        pltpu.make_async_copy(k_ref.at[h], kbuf.at[slot], ksem.at[slot]).start()
        pltpu.make_async_copy(v_ref.at[h], vbuf.at[slot], vsem.at[slot]).start()

    start_kv(0, 0)
    for h in range(HQ_PER):
        slot = h % 2
        pltpu.make_async_copy(k_ref.at[h], kbuf.at[slot], ksem.at[slot]).wait()
        pltpu.make_async_copy(v_ref.at[h], vbuf.at[slot], vsem.at[slot]).wait()
        if h + 1 < HQ_PER:
            start_kv(h + 1, (h + 1) % 2)

        qh = q_ref[:, h * DH:(h + 1) * DH].astype(BF)
        kh = kbuf[slot].astype(BF)
        s = lax.dot_general(qh, kh, (((1,), (1,)), ((), ())),
                            preferred_element_type=jnp.float32)
        s = s * SCALE + bias_ref[...]
        m = jnp.max(s, axis=-1, keepdims=True)
        w = jnp.exp(s - m)
        w = w / jnp.sum(w, axis=-1, keepdims=True)
        ctx_ref[:, h * DH:(h + 1) * DH] = jnp.dot(
            w.astype(BF), vbuf[slot].astype(BF),
            preferred_element_type=jnp.float32)

    partial = jnp.dot(ctx_ref[...].astype(BF), wo_ref[...].astype(BF),
                      preferred_element_type=jnp.float32)
    out_ref[...] = partial
    comm[0, :, :] = partial

    barrier = pltpu.get_barrier_semaphore()
    for nbr in (left, right):
        pl.semaphore_signal(barrier, inc=1, device_id=(nbr,),
                            device_id_type=pl.DeviceIdType.MESH)
    pl.semaphore_wait(barrier, 2)

    for h in range(N_DEV - 1):
        s_slot = h % 2
        r_slot = (h + 1) % 2
        if h >= 1:
            pl.semaphore_wait(credit_sem, 1)
        rdma = pltpu.make_async_remote_copy(
            src_ref=comm.at[s_slot],
            dst_ref=comm.at[r_slot],
            send_sem=send_sems.at[h],
            recv_sem=recv_sems.at[h],
            device_id=(right,),
            device_id_type=pl.DeviceIdType.MESH,
        )
        rdma.start()
        rdma.wait()
        out_ref[...] += comm[r_slot, :, :]
        if h <= N_DEV - 3:
            pl.semaphore_signal(credit_sem, inc=1, device_id=(left,),
                                device_id_type=pl.DeviceIdType.MESH)


def kernel(x, Wq, K_ext, V_ext, Wo):
    my = lax.axis_index("i")
    x2 = x.reshape(SQ, DM)
    k2 = K_ext.reshape(SKV, 128 * DH)
    v2 = V_ext.reshape(SKV, 128 * DH)
    kh = lax.dynamic_slice_in_dim(k2, my * HQ_PER * DH, HQ_PER * DH, axis=1)
    vh = lax.dynamic_slice_in_dim(v2, my * HQ_PER * DH, HQ_PER * DH, axis=1)
    kh = kh.reshape(SKV, HQ_PER, DH).transpose(1, 0, 2)
    vh = vh.reshape(SKV, HQ_PER, DH).transpose(1, 0, 2)

    out = pl.pallas_call(
        _body,
        out_shape=jax.ShapeDtypeStruct((SQ, DM), jnp.float32),
        in_specs=[
            pl.BlockSpec(memory_space=pltpu.VMEM),
            pl.BlockSpec(memory_space=pltpu.VMEM),
            pl.BlockSpec(memory_space=pltpu.ANY),
            pl.BlockSpec(memory_space=pltpu.ANY),
            pl.BlockSpec(memory_space=pltpu.VMEM),
        ],
        out_specs=pl.BlockSpec(memory_space=pltpu.VMEM),
        scratch_shapes=[
            pltpu.VMEM((SQ, DM), jnp.float32),
            pltpu.VMEM((SQ, SKV), jnp.float32),
            pltpu.VMEM((SQ, DM), jnp.float32),
            pltpu.VMEM((2, SKV, DH), jnp.float32),
            pltpu.VMEM((2, SKV, DH), jnp.float32),
            pltpu.VMEM((2, SQ, DM), jnp.float32),
            pltpu.SemaphoreType.DMA((2,)),
            pltpu.SemaphoreType.DMA((2,)),
            pltpu.SemaphoreType.DMA((N_DEV - 1,)),
            pltpu.SemaphoreType.DMA((N_DEV - 1,)),
            pltpu.SemaphoreType.REGULAR,
        ],
        compiler_params=pltpu.CompilerParams(collective_id=0),
    )(x2, Wq, kh, vh, Wo)
    return out.reshape(1, SQ, DM)


# baseline (device time: 807666 ns/iter reference)
import jax
import jax.numpy as jnp
from jax import lax
from jax.experimental import pallas as pl
from jax.experimental.pallas import tpu as pltpu

N_DEV = 16
HQ_PER = 8
DH = 128
SQ = 256
SKV = 4096
DM = 1024
SCALE = 0.08838834764831843
BF = jnp.bfloat16


def _body(x_ref, wq_ref, k_ref, v_ref, wo_ref, out_ref,
          q_ref, bias_ref, ctx_ref, kbuf, vbuf, comm,
          ksem, vsem, send_sems, recv_sems, credit_sem):
    my = lax.axis_index("i")
    left = lax.rem(my + N_DEV - 1, N_DEV)
    right = lax.rem(my + 1, N_DEV)

    q_ref[...] = jnp.dot(x_ref[...].astype(BF), wq_ref[...].astype(BF),
                         preferred_element_type=jnp.float32)

    qb = lax.broadcasted_iota(jnp.int32, (SQ, SKV), 0) // 64
    kb = lax.broadcasted_iota(jnp.int32, (SQ, SKV), 1) // 64
    mask = (qb == kb) | (kb == 0) | (lax.rem(qb + kb, 3) == 0)
    bias_ref[...] = jnp.where(mask, 0.0, -1e9)

    def start_kv(h, slot):
        pltpu.make_async_copy(k_ref.at[h], kbuf.at[slot], ksem.at[slot]).start()
        pltpu.make_async_copy(v_ref.at[h], vbuf.at[slot], vsem.at[slot]).start()

    start_kv(0, 0)
    for h in range(HQ_PER):
        slot = h % 2
        pltpu.make_async_copy(k_ref.at[h], kbuf.at[slot], ksem.at[slot]).wait()
        pltpu.make_async_copy(v_ref.at[h], vbuf.at[slot], vsem.at[slot]).wait()
        if h + 1 < HQ_PER:
            start_kv(h + 1, (h + 1) % 2)

        qh = q_ref[:, h * DH:(h + 1) * DH].astype(BF)
        kh = kbuf[slot].astype(BF)
        s = lax.dot_general(qh, kh, (((1,), (1,)), ((), ())),
                            preferred_element_type=jnp.float32)
        s = s * SCALE + bias_ref[...]
        m = jnp.max(s, axis=-1, keepdims=True)
        w = jnp.exp(s - m)
        w = w / jnp.sum(w, axis=-1, keepdims=True)
        ctx_ref[:, h * DH:(h + 1) * DH] = jnp.dot(
            w.astype(BF), vbuf[slot].astype(BF),
            preferred_element_type=jnp.float32)

    partial = jnp.dot(ctx_ref[...].astype(BF), wo_ref[...].astype(BF),
                      preferred_element_type=jnp.float32)
    out_ref[...] = partial
    comm[0, :, :] = partial

    barrier = pltpu.get_barrier_semaphore()
    for nbr in (left, right):
        pl.semaphore_signal(barrier, inc=1, device_id=(nbr,),
                            device_id_type=pl.DeviceIdType.MESH)
    pl.semaphore_wait(barrier, 2)

    for h in range(N_DEV - 1):
        s_slot = h % 2
        r_slot = (h + 1) % 2
        if h >= 1:
            pl.semaphore_wait(credit_sem, 1)
        rdma = pltpu.make_async_remote_copy(
            src_ref=comm.at[s_slot],
            dst_ref=comm.at[r_slot],
            send_sem=send_sems.at[h],
            recv_sem=recv_sems.at[h],
            device_id=(right,),
            device_id_type=pl.DeviceIdType.MESH,
        )
        rdma.start()
        rdma.wait()
        out_ref[...] += comm[r_slot, :, :]
        if h <= N_DEV - 3:
            pl.semaphore_signal(credit_sem, inc=1, device_id=(left,),
                                device_id_type=pl.DeviceIdType.MESH)


def kernel(x, Wq, K_ext, V_ext, Wo):
    my = lax.axis_index("i")
    x2 = x.reshape(SQ, DM)
    k2 = K_ext.reshape(SKV, 128 * DH)
    v2 = V_ext.reshape(SKV, 128 * DH)
    kh = lax.dynamic_slice_in_dim(k2, my * HQ_PER * DH, HQ_PER * DH, axis=1)
    vh = lax.dynamic_slice_in_dim(v2, my * HQ_PER * DH, HQ_PER * DH, axis=1)
    kh = kh.reshape(SKV, HQ_PER, DH).transpose(1, 0, 2)
    vh = vh.reshape(SKV, HQ_PER, DH).transpose(1, 0, 2)

    out = pl.pallas_call(
        _body,
        out_shape=jax.ShapeDtypeStruct((SQ, DM), jnp.float32),
        in_specs=[
            pl.BlockSpec(memory_space=pltpu.VMEM),
            pl.BlockSpec(memory_space=pltpu.VMEM),
            pl.BlockSpec(memory_space=pltpu.MemorySpace.HBM),
            pl.BlockSpec(memory_space=pltpu.MemorySpace.HBM),
            pl.BlockSpec(memory_space=pltpu.VMEM),
        ],
        out_specs=pl.BlockSpec(memory_space=pltpu.VMEM),
        scratch_shapes=[
            pltpu.VMEM((SQ, DM), jnp.float32),
            pltpu.VMEM((SQ, SKV), jnp.float32),
            pltpu.VMEM((SQ, DM), jnp.float32),
            pltpu.VMEM((2, SKV, DH), jnp.float32),
            pltpu.VMEM((2, SKV, DH), jnp.float32),
            pltpu.VMEM((2, SQ, DM), jnp.float32),
            pltpu.SemaphoreType.DMA((2,)),
            pltpu.SemaphoreType.DMA((2,)),
            pltpu.SemaphoreType.DMA((N_DEV - 1,)),
            pltpu.SemaphoreType.DMA((N_DEV - 1,)),
            pltpu.SemaphoreType.REGULAR,
        ],
        compiler_params=pltpu.CompilerParams(collective_id=0),
    )(x2, Wq, kh, vh, Wo)
    return out.reshape(1, SQ, DM)


# device time: 560400 ns/iter; 1.4412x vs baseline; 1.4412x over previous
import jax
import jax.numpy as jnp
from jax import lax
from jax.experimental import pallas as pl
from jax.experimental.pallas import tpu as pltpu

N_DEV = 16
HQ_PER = 8
DH = 128
SQ = 256
SKV = 4096
DM = 1024
SCALE = 0.08838834764831843
BF = jnp.bfloat16


def _body(x_ref, wq_ref, k_ref, v_ref, wo_ref, out_ref,
          q_ref, bias_ref, ctx_ref, kbuf, vbuf, comm,
          ksem, vsem, send_sems, recv_sems, credit_sem):
    my = lax.axis_index("i")
    left = lax.rem(my + N_DEV - 1, N_DEV)
    right = lax.rem(my + 1, N_DEV)

    q_ref[...] = jnp.dot(x_ref[...].astype(BF), wq_ref[...].astype(BF),
                         preferred_element_type=jnp.float32)

    qb = lax.broadcasted_iota(jnp.int32, (SQ, SKV), 0) // 64
    kb = lax.broadcasted_iota(jnp.int32, (SQ, SKV), 1) // 64
    mask = (qb == kb) | (kb == 0) | (lax.rem(qb + kb, 3) == 0)
    bias_ref[...] = jnp.where(mask, 0.0, -1e9)

    def start_kv(h, slot):
        pltpu.make_async_copy(k_ref.at[h], kbuf.at[slot], ksem.at[slot]).start()
        pltpu.make_async_copy(v_ref.at[h], vbuf.at[slot], vsem.at[slot]).start()

    start_kv(0, 0)
    for h in range(HQ_PER):
        slot = h % 2
        pltpu.make_async_copy(k_ref.at[h], kbuf.at[slot], ksem.at[slot]).wait()
        pltpu.make_async_copy(v_ref.at[h], vbuf.at[slot], vsem.at[slot]).wait()
        if h + 1 < HQ_PER:
            start_kv(h + 1, (h + 1) % 2)

        qh = q_ref[:, h * DH:(h + 1) * DH].astype(BF)
        kh = kbuf[slot].astype(BF)
        s = lax.dot_general(qh, kh, (((1,), (1,)), ((), ())),
                            preferred_element_type=jnp.float32)
        s = s * SCALE + bias_ref[...]
        m = jnp.max(s, axis=-1, keepdims=True)
        w = jnp.exp(s - m)
        w = w / jnp.sum(w, axis=-1, keepdims=True)
        ctx_ref[:, h * DH:(h + 1) * DH] = jnp.dot(
            w.astype(BF), vbuf[slot].astype(BF),
            preferred_element_type=jnp.float32)

    partial = jnp.dot(ctx_ref[...].astype(BF), wo_ref[...].astype(BF),
                      preferred_element_type=jnp.float32)
    out_ref[...] = partial
    comm[0, :, :] = partial

    RING = False
    if not RING:
        return
    barrier = pltpu.get_barrier_semaphore()
    for nbr in (left, right):
        pl.semaphore_signal(barrier, inc=1, device_id=(nbr,),
                            device_id_type=pl.DeviceIdType.MESH)
    pl.semaphore_wait(barrier, 2)

    for h in range(N_DEV - 1):
        s_slot = h % 2
        r_slot = (h + 1) % 2
        if h >= 1:
            pl.semaphore_wait(credit_sem, 1)
        rdma = pltpu.make_async_remote_copy(
            src_ref=comm.at[s_slot],
            dst_ref=comm.at[r_slot],
            send_sem=send_sems.at[h],
            recv_sem=recv_sems.at[h],
            device_id=(right,),
            device_id_type=pl.DeviceIdType.MESH,
        )
        rdma.start()
        rdma.wait()
        out_ref[...] += comm[r_slot, :, :]
        if h <= N_DEV - 3:
            pl.semaphore_signal(credit_sem, inc=1, device_id=(left,),
                                device_id_type=pl.DeviceIdType.MESH)


def kernel(x, Wq, K_ext, V_ext, Wo):
    my = lax.axis_index("i")
    x2 = x.reshape(SQ, DM)
    k2 = K_ext.reshape(SKV, 128 * DH)
    v2 = V_ext.reshape(SKV, 128 * DH)
    kh = lax.dynamic_slice_in_dim(k2, my * HQ_PER * DH, HQ_PER * DH, axis=1)
    vh = lax.dynamic_slice_in_dim(v2, my * HQ_PER * DH, HQ_PER * DH, axis=1)
    kh = kh.reshape(SKV, HQ_PER, DH).transpose(1, 0, 2)
    vh = vh.reshape(SKV, HQ_PER, DH).transpose(1, 0, 2)

    out = pl.pallas_call(
        _body,
        out_shape=jax.ShapeDtypeStruct((SQ, DM), jnp.float32),
        in_specs=[
            pl.BlockSpec(memory_space=pltpu.VMEM),
            pl.BlockSpec(memory_space=pltpu.VMEM),
            pl.BlockSpec(memory_space=pltpu.MemorySpace.HBM),
            pl.BlockSpec(memory_space=pltpu.MemorySpace.HBM),
            pl.BlockSpec(memory_space=pltpu.VMEM),
        ],
        out_specs=pl.BlockSpec(memory_space=pltpu.VMEM),
        scratch_shapes=[
            pltpu.VMEM((SQ, DM), jnp.float32),
            pltpu.VMEM((SQ, SKV), jnp.float32),
            pltpu.VMEM((SQ, DM), jnp.float32),
            pltpu.VMEM((2, SKV, DH), jnp.float32),
            pltpu.VMEM((2, SKV, DH), jnp.float32),
            pltpu.VMEM((2, SQ, DM), jnp.float32),
            pltpu.SemaphoreType.DMA((2,)),
            pltpu.SemaphoreType.DMA((2,)),
            pltpu.SemaphoreType.DMA((N_DEV - 1,)),
            pltpu.SemaphoreType.DMA((N_DEV - 1,)),
            pltpu.SemaphoreType.REGULAR,
        ],
    )(x2, Wq, kh, vh, Wo)
    return out.reshape(1, SQ, DM)


# device time: 449332 ns/iter; 1.7975x vs baseline; 1.2472x over previous
import jax
import jax.numpy as jnp
from jax import lax
from jax.experimental import pallas as pl
from jax.experimental.pallas import tpu as pltpu

N_DEV = 16
HQ_PER = 8
DH = 128
SQ = 256
SKV = 4096
DM = 1024
SCALE = 0.08838834764831843
BF = jnp.bfloat16

RS_BITS = (4, 8, 2, 1)
AG_BITS = (1, 2, 8, 4)
RBUF_OFF = (0, 128, 192, 224)


def _body(x_ref, wq_ref, k_ref, v_ref, wo_ref, out_ref,
          q_ref, bias_ref, ctx_ref, kall, vall, rbuf,
          kvsem, ssems, rsems):
    my = lax.axis_index("i")

    col0 = my * (HQ_PER * DH)
    kcp = pltpu.make_async_copy(
        k_ref.at[:, pl.ds(col0, HQ_PER * DH)], kall, kvsem.at[0])
    vcp = pltpu.make_async_copy(
        v_ref.at[:, pl.ds(col0, HQ_PER * DH)], vall, kvsem.at[1])
    kcp.start()
    vcp.start()

    q_ref[...] = jnp.dot(x_ref[...].astype(BF), wq_ref[...].astype(BF),
                         preferred_element_type=jnp.float32)

    qb = lax.broadcasted_iota(jnp.int32, (SQ, SKV), 0) // 64
    kb = lax.broadcasted_iota(jnp.int32, (SQ, SKV), 1) // 64
    mask = (qb == kb) | (kb == 0) | (lax.rem(qb + kb, 3) == 0)
    bias_ref[...] = jnp.where(mask, 0.0, -1e9)

    kcp.wait()
    vcp.wait()

    for h in range(HQ_PER):
        qh = q_ref[:, h * DH:(h + 1) * DH].astype(BF)
        kh = kall[:, h * DH:(h + 1) * DH].astype(BF)
        s = lax.dot_general(qh, kh, (((1,), (1,)), ((), ())),
                            preferred_element_type=jnp.float32)
        s = s * SCALE + bias_ref[...]
        m = jnp.max(s, axis=-1, keepdims=True)
        w = jnp.exp(s - m)
        w = w / jnp.sum(w, axis=-1, keepdims=True)
        ctx_ref[:, h * DH:(h + 1) * DH] = jnp.dot(
            w.astype(BF), vall[:, h * DH:(h + 1) * DH].astype(BF),
            preferred_element_type=jnp.float32)

    out_ref[...] = jnp.dot(ctx_ref[...].astype(BF), wo_ref[...].astype(BF),
                           preferred_element_type=jnp.float32)

    barrier = pltpu.get_barrier_semaphore()
    for bit in (1, 2, 4, 8):
        pl.semaphore_signal(barrier, inc=1, device_id=(my ^ bit,),
                            device_id_type=pl.DeviceIdType.MESH)
    pl.semaphore_wait(barrier, 4)

    lo = jnp.int32(0)
    sz = SQ
    for s, bit in enumerate(RS_BITS):
        half = sz // 2
        partner = my ^ bit
        upper = (my & bit) != 0
        send_lo = lo + jnp.where(upper, 0, half)
        keep_lo = lo + jnp.where(upper, half, 0)
        rdma = pltpu.make_async_remote_copy(
            src_ref=out_ref.at[pl.ds(send_lo, half), :],
            dst_ref=rbuf.at[pl.ds(RBUF_OFF[s], half), :],
            send_sem=ssems.at[s],
            recv_sem=rsems.at[s],
            device_id=(partner,),
            device_id_type=pl.DeviceIdType.MESH,
        )
        rdma.start()
        rdma.wait()
        out_ref[pl.ds(keep_lo, half), :] += rbuf[pl.ds(RBUF_OFF[s], half), :]
        lo = keep_lo
        sz = half

    for s, bit in enumerate(AG_BITS):
        partner = my ^ bit
        rdma = pltpu.make_async_remote_copy(
            src_ref=out_ref.at[pl.ds(lo, sz), :],
            dst_ref=out_ref.at[pl.ds(lo, sz), :],
            send_sem=ssems.at[4 + s],
            recv_sem=rsems.at[4 + s],
            device_id=(partner,),
            device_id_type=pl.DeviceIdType.MESH,
        )
        rdma.start()
        rdma.wait()
        lo = lo - jnp.where((my & bit) != 0, sz, 0)
        sz = sz * 2


def kernel(x, Wq, K_ext, V_ext, Wo):
    x2 = x.reshape(SQ, DM)
    k2 = K_ext.reshape(SKV, 128 * DH)
    v2 = V_ext.reshape(SKV, 128 * DH)

    out = pl.pallas_call(
        _body,
        out_shape=jax.ShapeDtypeStruct((SQ, DM), jnp.float32),
        in_specs=[
            pl.BlockSpec(memory_space=pltpu.VMEM),
            pl.BlockSpec(memory_space=pltpu.VMEM),
            pl.BlockSpec(memory_space=pltpu.MemorySpace.HBM),
            pl.BlockSpec(memory_space=pltpu.MemorySpace.HBM),
            pl.BlockSpec(memory_space=pltpu.VMEM),
        ],
        out_specs=pl.BlockSpec(memory_space=pltpu.VMEM),
        scratch_shapes=[
            pltpu.VMEM((SQ, DM), jnp.float32),
            pltpu.VMEM((SQ, SKV), jnp.float32),
            pltpu.VMEM((SQ, DM), jnp.float32),
            pltpu.VMEM((SKV, HQ_PER * DH), jnp.float32),
            pltpu.VMEM((SKV, HQ_PER * DH), jnp.float32),
            pltpu.VMEM((SQ, DM), jnp.float32),
            pltpu.SemaphoreType.DMA((2,)),
            pltpu.SemaphoreType.DMA((8,)),
            pltpu.SemaphoreType.DMA((8,)),
        ],
        compiler_params=pltpu.CompilerParams(
            collective_id=0, vmem_limit_bytes=56 * 1024 * 1024),
    )(x2, Wq, k2, v2, Wo)
    return out.reshape(1, SQ, DM)


# device time: 75536 ns/iter; 10.6925x vs baseline; 5.9486x over previous
import jax
import jax.numpy as jnp
from jax import lax
from jax.experimental import pallas as pl
from jax.experimental.pallas import tpu as pltpu

N_DEV = 16
HQ_PER = 8
DH = 128
SQ = 256
SKV = 4096
DM = 1024
SCALE = 0.08838834764831843
BF = jnp.bfloat16

RS_BITS = (4, 8, 2, 1)
AG_BITS = (1, 2, 8, 4)
RBUF_OFF = (0, 128, 192, 224)


def _body(x_ref, wq_ref, k_ref, v_ref, wo_ref, out_ref,
          q_ref, bias_ref, ctx_ref, kall, vall, rbuf,
          kvsem, ssems, rsems):
    my = lax.axis_index("i")

    def kv_copies(h):
        gh = my * HQ_PER + h
        return (
            pltpu.make_async_copy(k_ref.at[0, :, gh, :],
                                  kall.at[:, pl.ds(h * DH, DH)], kvsem.at[0, h]),
            pltpu.make_async_copy(v_ref.at[0, :, gh, :],
                                  vall.at[:, pl.ds(h * DH, DH)], kvsem.at[1, h]),
        )

    for h in range(HQ_PER):
        kcp, vcp = kv_copies(h)
        kcp.start()
        vcp.start()

    q_ref[...] = jnp.dot(x_ref[...].astype(BF), wq_ref[...].astype(BF),
                         preferred_element_type=jnp.float32)

    qb = lax.broadcasted_iota(jnp.int32, (SQ, SKV), 0) // 64
    kb = lax.broadcasted_iota(jnp.int32, (SQ, SKV), 1) // 64
    mask = (qb == kb) | (kb == 0) | (lax.rem(qb + kb, 3) == 0)
    bias_ref[...] = jnp.where(mask, 0.0, -1e9)

    for h in range(HQ_PER):
        kcp, vcp = kv_copies(h)
        kcp.wait()
        vcp.wait()
        qh = q_ref[:, h * DH:(h + 1) * DH].astype(BF)
        kh = kall[:, h * DH:(h + 1) * DH].astype(BF)
        s = lax.dot_general(qh, kh, (((1,), (1,)), ((), ())),
                            preferred_element_type=jnp.float32)
        s = s * SCALE + bias_ref[...]
        m = jnp.max(s, axis=-1, keepdims=True)
        w = jnp.exp(s - m)
        w = w / jnp.sum(w, axis=-1, keepdims=True)
        ctx_ref[:, h * DH:(h + 1) * DH] = jnp.dot(
            w.astype(BF), vall[:, h * DH:(h + 1) * DH].astype(BF),
            preferred_element_type=jnp.float32)

    out_ref[...] = jnp.dot(ctx_ref[...].astype(BF), wo_ref[...].astype(BF),
                           preferred_element_type=jnp.float32)

    barrier = pltpu.get_barrier_semaphore()
    for bit in (1, 2, 4, 8):
        pl.semaphore_signal(barrier, inc=1, device_id=(my ^ bit,),
                            device_id_type=pl.DeviceIdType.MESH)
    pl.semaphore_wait(barrier, 4)

    lo = jnp.int32(0)
    sz = SQ
    for s, bit in enumerate(RS_BITS):
        half = sz // 2
        partner = my ^ bit
        upper = (my & bit) != 0
        send_lo = lo + jnp.where(upper, 0, half)
        keep_lo = lo + jnp.where(upper, half, 0)
        rdma = pltpu.make_async_remote_copy(
            src_ref=out_ref.at[pl.ds(send_lo, half), :],
            dst_ref=rbuf.at[pl.ds(RBUF_OFF[s], half), :],
            send_sem=ssems.at[s],
            recv_sem=rsems.at[s],
            device_id=(partner,),
            device_id_type=pl.DeviceIdType.MESH,
        )
        rdma.start()
        rdma.wait()
        out_ref[pl.ds(keep_lo, half), :] += rbuf[pl.ds(RBUF_OFF[s], half), :]
        lo = keep_lo
        sz = half

    for s, bit in enumerate(AG_BITS):
        partner = my ^ bit
        rdma = pltpu.make_async_remote_copy(
            src_ref=out_ref.at[pl.ds(lo, sz), :],
            dst_ref=out_ref.at[pl.ds(lo, sz), :],
            send_sem=ssems.at[4 + s],
            recv_sem=rsems.at[4 + s],
            device_id=(partner,),
            device_id_type=pl.DeviceIdType.MESH,
        )
        rdma.start()
        rdma.wait()
        lo = lo - jnp.where((my & bit) != 0, sz, 0)
        sz = sz * 2


def kernel(x, Wq, K_ext, V_ext, Wo):
    x2 = x.reshape(SQ, DM)

    out = pl.pallas_call(
        _body,
        out_shape=jax.ShapeDtypeStruct((SQ, DM), jnp.float32),
        in_specs=[
            pl.BlockSpec(memory_space=pltpu.VMEM),
            pl.BlockSpec(memory_space=pltpu.VMEM),
            pl.BlockSpec(memory_space=pltpu.MemorySpace.HBM),
            pl.BlockSpec(memory_space=pltpu.MemorySpace.HBM),
            pl.BlockSpec(memory_space=pltpu.VMEM),
        ],
        out_specs=pl.BlockSpec(memory_space=pltpu.VMEM),
        scratch_shapes=[
            pltpu.VMEM((SQ, DM), jnp.float32),
            pltpu.VMEM((SQ, SKV), jnp.float32),
            pltpu.VMEM((SQ, DM), jnp.float32),
            pltpu.VMEM((SKV, HQ_PER * DH), jnp.float32),
            pltpu.VMEM((SKV, HQ_PER * DH), jnp.float32),
            pltpu.VMEM((SQ, DM), jnp.float32),
            pltpu.SemaphoreType.DMA((2, HQ_PER)),
            pltpu.SemaphoreType.DMA((8,)),
            pltpu.SemaphoreType.DMA((8,)),
        ],
        compiler_params=pltpu.CompilerParams(
            collective_id=0, vmem_limit_bytes=56 * 1024 * 1024),
    )(x2, Wq, K_ext, V_ext, Wo)
    return out.reshape(1, SQ, DM)


# device time: 55336 ns/iter; 14.5957x vs baseline; 1.3650x over previous
import os
import jax
import jax.numpy as jnp
from jax import lax
from jax.experimental import pallas as pl
from jax.experimental.pallas import tpu as pltpu

N_DEV = 16
HQ_PER = 8
DH = 128
SQ = 256
SKV = 4096
DM = 1024
CH = SQ // N_DEV
SCALE = 0.08838834764831843
BF = jnp.bfloat16


def _body(x_ref, wq_ref, k_ref, v_ref, wo_ref, out_ref,
          q_ref, bias_ref, ctx_ref, kall, vall, rbuf,
          kvsem, ssems1, ssems2, rsems1, rsems2):
    my = lax.axis_index("i")

    def kv_copies(h):
        gh = my * HQ_PER + h
        return (
            pltpu.make_async_copy(k_ref.at[0, :, gh, :],
                                  kall.at[:, pl.ds(h * DH, DH)], kvsem.at[0, h]),
            pltpu.make_async_copy(v_ref.at[0, :, gh, :],
                                  vall.at[:, pl.ds(h * DH, DH)], kvsem.at[1, h]),
        )

    for h in range(HQ_PER):
        kcp, vcp = kv_copies(h)
        kcp.start()
        vcp.start()

    q_ref[...] = jnp.dot(x_ref[...].astype(BF), wq_ref[...].astype(BF),
                         preferred_element_type=jnp.float32)

    qb = lax.broadcasted_iota(jnp.int32, (SQ, SKV), 0) // 64
    kb = lax.broadcasted_iota(jnp.int32, (SQ, SKV), 1) // 64
    mask = (qb == kb) | (kb == 0) | (lax.rem(qb + kb, 3) == 0)
    bias_ref[...] = jnp.where(mask, 0.0, -1e9)

    for h in range(HQ_PER):
        kcp, vcp = kv_copies(h)
        kcp.wait()
        vcp.wait()
        qh = q_ref[:, h * DH:(h + 1) * DH].astype(BF)
        kh = kall[:, h * DH:(h + 1) * DH].astype(BF)
        s = lax.dot_general(qh, kh, (((1,), (1,)), ((), ())),
                            preferred_element_type=jnp.float32)
        w = jnp.exp(s * SCALE + bias_ref[...])
        r = 1.0 / jnp.sum(w, axis=-1, keepdims=True)
        ctx_ref[:, h * DH:(h + 1) * DH] = jnp.dot(
            w.astype(BF), vall[:, h * DH:(h + 1) * DH].astype(BF),
            preferred_element_type=jnp.float32) * r

    out_ref[...] = jnp.dot(ctx_ref[...].astype(BF), wo_ref[...].astype(BF),
                           preferred_element_type=jnp.float32)

    if os.environ.get('NOBFLY') == '1':
        return

    barrier = pltpu.get_barrier_semaphore()
    for d in range(1, N_DEV):
        pl.semaphore_signal(barrier, inc=1,
                            device_id=(lax.rem(my + d, N_DEV),),
                            device_id_type=pl.DeviceIdType.MESH)
    pl.semaphore_wait(barrier, N_DEV - 1)

    p1 = []
    for d in range(1, N_DEV):
        dst = lax.rem(my + d, N_DEV)
        rdma = pltpu.make_async_remote_copy(
            src_ref=out_ref.at[pl.ds(dst * CH, CH), :],
            dst_ref=rbuf.at[my],
            send_sem=ssems1.at[d - 1],
            recv_sem=rsems1.at[my],
            device_id=(dst,),
            device_id_type=pl.DeviceIdType.MESH,
        )
        rdma.start()
        p1.append(rdma)

    own = my * CH
    acc = out_ref[pl.ds(own, CH), :]
    for d in range(1, N_DEV):
        src = lax.rem(my - d + N_DEV, N_DEV)
        recv = pltpu.make_async_remote_copy(
            src_ref=rbuf.at[src], dst_ref=rbuf.at[src],
            send_sem=ssems1.at[0], recv_sem=rsems1.at[src],
            device_id=(src,), device_id_type=pl.DeviceIdType.MESH,
        )
        recv.wait_recv()
        acc = acc + rbuf[src]
    out_ref[pl.ds(own, CH), :] = acc

    p2 = []
    for d in range(1, N_DEV):
        dst = lax.rem(my + d, N_DEV)
        rdma = pltpu.make_async_remote_copy(
            src_ref=out_ref.at[pl.ds(own, CH), :],
            dst_ref=out_ref.at[pl.ds(own, CH), :],
            send_sem=ssems2.at[d - 1],
            recv_sem=rsems2.at[my],
            device_id=(dst,),
            device_id_type=pl.DeviceIdType.MESH,
        )
        rdma.start()
        p2.append(rdma)

    for d in range(1, N_DEV):
        src = lax.rem(my - d + N_DEV, N_DEV)
        recv = pltpu.make_async_remote_copy(
            src_ref=out_ref.at[pl.ds(src * CH, CH), :],
            dst_ref=out_ref.at[pl.ds(src * CH, CH), :],
            send_sem=ssems2.at[0], recv_sem=rsems2.at[src],
            device_id=(src,), device_id_type=pl.DeviceIdType.MESH,
        )
        recv.wait_recv()

    for rdma in p1:
        rdma.wait_send()
    for rdma in p2:
        rdma.wait_send()


def kernel(x, Wq, K_ext, V_ext, Wo):
    x2 = x.reshape(SQ, DM)

    out = pl.pallas_call(
        _body,
        out_shape=jax.ShapeDtypeStruct((SQ, DM), jnp.float32),
        in_specs=[
            pl.BlockSpec(memory_space=pltpu.VMEM),
            pl.BlockSpec(memory_space=pltpu.VMEM),
            pl.BlockSpec(memory_space=pltpu.MemorySpace.HBM),
            pl.BlockSpec(memory_space=pltpu.MemorySpace.HBM),
            pl.BlockSpec(memory_space=pltpu.VMEM),
        ],
        out_specs=pl.BlockSpec(memory_space=pltpu.VMEM),
        scratch_shapes=[
            pltpu.VMEM((SQ, DM), jnp.float32),
            pltpu.VMEM((SQ, SKV), jnp.float32),
            pltpu.VMEM((SQ, DM), jnp.float32),
            pltpu.VMEM((SKV, HQ_PER * DH), jnp.float32),
            pltpu.VMEM((SKV, HQ_PER * DH), jnp.float32),
            pltpu.VMEM((N_DEV, CH, DM), jnp.float32),
            pltpu.SemaphoreType.DMA((2, HQ_PER)),
            pltpu.SemaphoreType.DMA((N_DEV - 1,)),
            pltpu.SemaphoreType.DMA((N_DEV - 1,)),
            pltpu.SemaphoreType.DMA((N_DEV,)),
            pltpu.SemaphoreType.DMA((N_DEV,)),
        ],
        compiler_params=(pltpu.CompilerParams(vmem_limit_bytes=56 * 1024 * 1024)
                         if os.environ.get('NOBFLY') == '1' else
                         pltpu.CompilerParams(
                             collective_id=0, vmem_limit_bytes=56 * 1024 * 1024)),
    )(x2, Wq, K_ext, V_ext, Wo)
    return out.reshape(1, SQ, DM)


# device time: 27206 ns/iter; 29.6871x vs baseline; 2.0340x over previous
import os
import jax
import jax.numpy as jnp
from jax import lax
from jax.experimental import pallas as pl
from jax.experimental.pallas import tpu as pltpu

N_DEV = 16
HQ_PER = 8
DH = 128
SQ = 256
SKV = 4096
DM = 1024
CH = SQ // N_DEV
HALF = SQ // 2
SCALE = 0.08838834764831843
BF = jnp.bfloat16

_NO_AR = os.environ.get('NOBFLY') == '1'


def _body(x_ref, wq_ref, k_ref, v_ref, wo_ref, out_ref,
          q_ref, bias_ref, ctx_ref, kall, vall, rbuf,
          kvsem, ssems1, ssems2, rsems1, rsems2):
    my = lax.axis_index("i")

    if not _NO_AR:
        barrier = pltpu.get_barrier_semaphore()
        for d in range(1, N_DEV):
            pl.semaphore_signal(barrier, inc=1,
                                device_id=(lax.rem(my + d, N_DEV),),
                                device_id_type=pl.DeviceIdType.MESH)
        pl.semaphore_wait(barrier, N_DEV - 1)

    def kv_copies(h):
        gh = my * HQ_PER + h
        return (
            pltpu.make_async_copy(k_ref.at[0, :, gh, :],
                                  kall.at[:, pl.ds(h * DH, DH)], kvsem.at[0, h]),
            pltpu.make_async_copy(v_ref.at[0, :, gh, :],
                                  vall.at[:, pl.ds(h * DH, DH)], kvsem.at[1, h]),
        )

    for h in range(HQ_PER):
        kcp, vcp = kv_copies(h)
        kcp.start()
        vcp.start()

    def p1_rdma(c):
        return pltpu.make_async_remote_copy(
            src_ref=out_ref.at[pl.ds(c * CH, CH), :],
            dst_ref=rbuf.at[my],
            send_sem=ssems1.at[c],
            recv_sem=rsems1.at[my],
            device_id=(c,),
            device_id_type=pl.DeviceIdType.MESH,
        )

    q_ref[...] = jnp.dot(x_ref[...].astype(BF), wq_ref[...].astype(BF),
                         preferred_element_type=jnp.float32)

    qb = lax.broadcasted_iota(jnp.int32, (SQ, SKV), 0) // 64
    kb = lax.broadcasted_iota(jnp.int32, (SQ, SKV), 1) // 64
    mask = (qb == kb) | (kb == 0) | (lax.rem(qb + kb, 3) == 0)
    bias_ref[...] = jnp.where(mask, 0.0, -1e9)

    for half in range(2):
        r0 = half * HALF
        rows = pl.ds(r0, HALF)
        for h in range(HQ_PER):
            if half == 0:
                kcp, vcp = kv_copies(h)
                kcp.wait()
                vcp.wait()
            qh = q_ref[rows, h * DH:(h + 1) * DH].astype(BF)
            kh = kall[:, h * DH:(h + 1) * DH].astype(BF)
            s = lax.dot_general(qh, kh, (((1,), (1,)), ((), ())),
                                preferred_element_type=jnp.float32)
            w = jnp.exp(s * SCALE + bias_ref[rows, :])
            r = 1.0 / jnp.sum(w, axis=-1, keepdims=True)
            ctx_ref[rows, h * DH:(h + 1) * DH] = jnp.dot(
                w.astype(BF), vall[:, h * DH:(h + 1) * DH].astype(BF),
                preferred_element_type=jnp.float32) * r

        out_ref[rows, :] = jnp.dot(
            ctx_ref[rows, :].astype(BF), wo_ref[...].astype(BF),
            preferred_element_type=jnp.float32)
        if not _NO_AR:
            for c in range(half * (N_DEV // 2), (half + 1) * (N_DEV // 2)):
                @pl.when(my != c)
                def _(c=c):
                    p1_rdma(c).start()

    if _NO_AR:
        return

    own = my * CH
    acc = out_ref[pl.ds(own, CH), :]
    for d in range(1, N_DEV):
        src = lax.rem(my - d + N_DEV, N_DEV)
        recv = pltpu.make_async_remote_copy(
            src_ref=rbuf.at[src], dst_ref=rbuf.at[src],
            send_sem=ssems1.at[0], recv_sem=rsems1.at[src],
            device_id=(src,), device_id_type=pl.DeviceIdType.MESH,
        )
        recv.wait_recv()
        acc = acc + rbuf[src]
    out_ref[pl.ds(own, CH), :] = acc

    p2 = []
    for d in range(1, N_DEV):
        dst = lax.rem(my + d, N_DEV)
        rdma = pltpu.make_async_remote_copy(
            src_ref=out_ref.at[pl.ds(own, CH), :],
            dst_ref=out_ref.at[pl.ds(own, CH), :],
            send_sem=ssems2.at[d - 1],
            recv_sem=rsems2.at[my],
            device_id=(dst,),
            device_id_type=pl.DeviceIdType.MESH,
        )
        rdma.start()
        p2.append(rdma)

    for d in range(1, N_DEV):
        src = lax.rem(my - d + N_DEV, N_DEV)
        recv = pltpu.make_async_remote_copy(
            src_ref=out_ref.at[pl.ds(src * CH, CH), :],
            dst_ref=out_ref.at[pl.ds(src * CH, CH), :],
            send_sem=ssems2.at[0], recv_sem=rsems2.at[src],
            device_id=(src,), device_id_type=pl.DeviceIdType.MESH,
        )
        recv.wait_recv()

    for c in range(N_DEV):
        @pl.when(my != c)
        def _(c=c):
            p1_rdma(c).wait_send()
    for rdma in p2:
        rdma.wait_send()


def kernel(x, Wq, K_ext, V_ext, Wo):
    x2 = x.reshape(SQ, DM)

    out = pl.pallas_call(
        _body,
        out_shape=jax.ShapeDtypeStruct((SQ, DM), jnp.float32),
        in_specs=[
            pl.BlockSpec(memory_space=pltpu.VMEM),
            pl.BlockSpec(memory_space=pltpu.VMEM),
            pl.BlockSpec(memory_space=pltpu.MemorySpace.HBM),
            pl.BlockSpec(memory_space=pltpu.MemorySpace.HBM),
            pl.BlockSpec(memory_space=pltpu.VMEM),
        ],
        out_specs=pl.BlockSpec(memory_space=pltpu.VMEM),
        scratch_shapes=[
            pltpu.VMEM((SQ, DM), jnp.float32),
            pltpu.VMEM((SQ, SKV), jnp.float32),
            pltpu.VMEM((SQ, DM), jnp.float32),
            pltpu.VMEM((SKV, HQ_PER * DH), jnp.float32),
            pltpu.VMEM((SKV, HQ_PER * DH), jnp.float32),
            pltpu.VMEM((N_DEV, CH, DM), jnp.float32),
            pltpu.SemaphoreType.DMA((2, HQ_PER)),
            pltpu.SemaphoreType.DMA((N_DEV,)),
            pltpu.SemaphoreType.DMA((N_DEV - 1,)),
            pltpu.SemaphoreType.DMA((N_DEV,)),
            pltpu.SemaphoreType.DMA((N_DEV,)),
        ],
        compiler_params=(pltpu.CompilerParams(vmem_limit_bytes=60 * 1024 * 1024)
                         if _NO_AR else
                         pltpu.CompilerParams(
                             collective_id=0, vmem_limit_bytes=60 * 1024 * 1024)),
    )(x2, Wq, K_ext, V_ext, Wo)
    return out.reshape(1, SQ, DM)
